# baseline (speedup 1.0000x reference)
import functools

import jax
import jax.numpy as jnp
from jax import lax
from jax.experimental import pallas as pl
from jax.experimental.pallas import tpu as pltpu
from jax.experimental.pallas import tpu_sc as plsc

ITEM_NUM = 1000000
EMB_SIZE = 64
MAX_LEN = 200
BATCH = 4096

_INFO = plsc.get_sparse_core_info()
_NC = _INFO.num_cores
_NS = _INFO.num_subcores
_NW = _NC * _NS
_SEQ_PER_W = BATCH // _NW
_HALF = MAX_LEN // 2
_LANES = 16
_VPR = EMB_SIZE // _LANES


def _make_kernel():
    mesh = plsc.VectorSubcoreMesh(core_axis_name="c", subcore_axis_name="s")

    @functools.partial(
        pl.kernel,
        out_type=jax.ShapeDtypeStruct((BATCH * MAX_LEN, EMB_SIZE), jnp.float32),
        mesh=mesh,
        scratch_types=[
            pltpu.VMEM((2, _HALF), jnp.int32),
            pltpu.VMEM((MAX_LEN, EMB_SIZE), jnp.float32),
            pltpu.VMEM((MAX_LEN, EMB_SIZE), jnp.float32),
            pltpu.SemaphoreType.DMA,
        ],
        compiler_params=pltpu.CompilerParams(use_tc_tiling_on_sc=False),
    )
    def emb_kernel(seqs_hbm, table_hbm, pos_hbm, out_hbm, idx_v, pos_v, rows_v, sem):
        wid = lax.axis_index("s") * _NC + lax.axis_index("c")
        base_seq = wid * _SEQ_PER_W

        pltpu.sync_copy(pos_hbm, pos_v)

        def per_seq(s, carry):
            b = base_seq + s
            pltpu.sync_copy(seqs_hbm.at[b], idx_v)
            d0 = pltpu.async_copy(
                table_hbm.at[idx_v.at[0]], rows_v.at[pl.ds(0, _HALF)], sem)
            d1 = pltpu.async_copy(
                table_hbm.at[idx_v.at[1]], rows_v.at[pl.ds(_HALF, _HALF)], sem)
            d0.wait()
            d1.wait()

            def add_row(r, c):
                for j in range(_VPR):
                    sl = pl.ds(j * _LANES, _LANES)
                    rows_v[r, sl] = rows_v[r, sl] + pos_v[r, sl]
                return c

            lax.fori_loop(0, MAX_LEN, add_row, 0)
            pltpu.sync_copy(rows_v, out_hbm.at[pl.ds(b * MAX_LEN, MAX_LEN)])
            return carry

        lax.fori_loop(0, _SEQ_PER_W, per_seq, 0)

    return emb_kernel


_EMB_KERNEL = _make_kernel()


def kernel(batch_seqs, token_table, pos_table):
    seqs = batch_seqs.astype(jnp.int32).reshape(BATCH, 2, _HALF)
    out = _EMB_KERNEL(seqs, token_table, pos_table)
    return out.reshape(BATCH, MAX_LEN, EMB_SIZE)
